# baseline (speedup 1.0000x reference)
import functools

import jax
import jax.numpy as jnp
from jax import lax
from jax.experimental import pallas as pl
from jax.experimental.pallas import tpu as pltpu
from jax.experimental.pallas import tpu_sc as plsc

F32 = jnp.float32


def _sc_info():
    try:
        info = plsc.get_sparse_core_info()
        return info.num_cores, info.num_subcores, info.num_lanes
    except Exception:
        return 2, 16, 16


def _make_deg_kernel(n_pad, nc, ns, n_chunks, K):
    NW = nc * ns
    ZR = n_pad // ns
    mesh = plsc.VectorSubcoreMesh(
        core_axis_name="c", subcore_axis_name="s", num_cores=nc,
        num_subcores=ns)

    @functools.partial(
        pl.kernel,
        out_type=jax.ShapeDtypeStruct((nc, n_pad), F32),
        mesh=mesh,
        scratch_types=[
            pltpu.VMEM((n_chunks, K), jnp.int32),
            pltpu.VMEM((K,), F32),
            pltpu.VMEM_SHARED((n_pad,), F32),
        ],
    )
    def deg_kernel(col3_hbm, zeros1_hbm, out_hbm, col2d, ones_v, deg_sh):
        cid = lax.axis_index("c")
        sid = lax.axis_index("s")
        wid = sid * nc + cid
        pltpu.sync_copy(zeros1_hbm, deg_sh.at[pl.ds(sid * ZR, ZR)])
        pltpu.sync_copy(col3_hbm.at[wid], col2d)
        for j in range(K // 16):
            ones_v[pl.ds(j * 16, 16)] = jnp.full((16,), 1.0, dtype=F32)
        plsc.subcore_barrier()

        def chunk(i, carry):
            pltpu.sync_copy(ones_v, deg_sh.at[col2d.at[i]], add=True)
            return carry

        lax.fori_loop(0, n_chunks, chunk, 0)
        plsc.subcore_barrier()
        pltpu.sync_copy(deg_sh.at[pl.ds(sid * ZR, ZR)],
                        out_hbm.at[cid, pl.ds(sid * ZR, ZR)])

    return deg_kernel


def _make_agg_kernel(n_pad, d, nc, ns, n_chunks, K):
    ZR = n_pad // ns
    mesh = plsc.VectorSubcoreMesh(
        core_axis_name="c", subcore_axis_name="s", num_cores=nc,
        num_subcores=ns)

    S = 2
    assert n_chunks % S == 0
    n_iters = n_chunks // S

    @functools.partial(
        pl.kernel,
        out_type=jax.ShapeDtypeStruct((nc, n_pad, d), F32),
        mesh=mesh,
        scratch_types=(
            [pltpu.VMEM((2, K), jnp.int32)] * S
            + [pltpu.VMEM((K, d), F32)] * S
            + [pltpu.SemaphoreType.DMA] * (3 * S)
            + [pltpu.VMEM_SHARED((n_pad, d), F32)]
        ),
    )
    def agg_kernel(h_hbm, idx4_hbm, zeros2_hbm, out_hbm, *refs):
        idx = refs[0:S]
        buf = refs[S:2 * S]
        isem = refs[2 * S:3 * S]
        gsem = refs[3 * S:4 * S]
        ssem = refs[4 * S:5 * S]
        agg_sh = refs[5 * S]
        cid = lax.axis_index("c")
        sid = lax.axis_index("s")
        wid = sid * nc + cid
        pltpu.sync_copy(zeros2_hbm, agg_sh.at[pl.ds(sid * ZR, ZR)])
        h_tbl = h_hbm
        plsc.subcore_barrier()

        for t in range(S):
            pltpu.async_copy(idx4_hbm.at[wid, t], idx[t], isem[t])
        for t in range(S):
            pltpu.make_async_copy(idx4_hbm.at[wid, t], idx[t], isem[t]).wait()
            pltpu.async_copy(h_tbl.at[idx[t].at[0]], buf[t], gsem[t])

        def trip(j, carry):
            i0 = S * j
            for t in range(S):
                pltpu.make_async_copy(
                    h_tbl.at[idx[t].at[0]], buf[t], gsem[t]).wait()
                pltpu.async_copy(
                    buf[t], agg_sh.at[idx[t].at[1]], ssem[t], add=True)
            for t in range(S):
                i2 = i0 + S + t

                @pl.when(i2 < n_chunks)
                def _(t=t, i2=i2):
                    pltpu.make_async_copy(
                        buf[t], agg_sh.at[idx[t].at[1]], ssem[t]).wait()
                    pltpu.async_copy(idx4_hbm.at[wid, i2], idx[t], isem[t])
                    pltpu.make_async_copy(
                        idx4_hbm.at[wid, i2], idx[t], isem[t]).wait()
                    pltpu.async_copy(h_tbl.at[idx[t].at[0]], buf[t], gsem[t])

            return carry

        lax.fori_loop(0, n_iters, trip, 0)
        for t in range(S):
            pltpu.make_async_copy(
                buf[t], agg_sh.at[idx[t].at[1]], ssem[t]).wait()
        plsc.subcore_barrier()
        pltpu.sync_copy(agg_sh.at[pl.ds(sid * ZR, ZR)],
                        out_hbm.at[cid, pl.ds(sid * ZR, ZR)])

    return agg_kernel


def _tc1_body(x_ref, w_ref, degp_ref, o_ref):
    deg = degp_ref[0] + degp_ref[1] + 1.0
    dinv = lax.rsqrt(deg)
    h = jnp.dot(x_ref[...], w_ref[...], preferred_element_type=F32)
    o_ref[...] = h * dinv


def _tc2_body(aggp_ref, hp_ref, degp_ref, b_ref, w_ref, o_ref):
    deg = degp_ref[0] + degp_ref[1] + 1.0
    dinv = lax.rsqrt(deg)
    a = aggp_ref[0] + aggp_ref[1] + hp_ref[...]
    z = a * dinv + b_ref[...]
    x1 = jnp.maximum(z, 0.0)
    o_ref[...] = jnp.dot(x1, w_ref[...], preferred_element_type=F32) * dinv


def _tc3_body(d_out, aggp_ref, hp_ref, degp_ref, b_ref, o_ref):
    deg = degp_ref[0] + degp_ref[1] + 1.0
    dinv = lax.rsqrt(deg)
    zf = (aggp_ref[0] + aggp_ref[1] + hp_ref[...]) * dinv
    z = zf[:, :d_out] + b_ref[...]
    m = jnp.max(z, axis=-1, keepdims=True)
    lse = m + jnp.log(jnp.sum(jnp.exp(z - m), axis=-1, keepdims=True))
    o_ref[...] = z - lse


def kernel(x, edge_index, W1, b1, W2, b2):
    N, D_in = x.shape
    D_h = W1.shape[1]
    D_out = W2.shape[1]
    E = edge_index.shape[1]

    NC, NS, _L = _sc_info()
    NW = NC * NS
    K = 128
    BR = 1024
    SD = 2

    n_pad = ((N + NS * 8 - 1) // (NS * 8)) * (NS * 8)
    if n_pad % BR:
        n_pad = ((n_pad + BR - 1) // BR) * BR
    ZR = n_pad // NS
    e_pad = ((E + SD * NW * K - 1) // (SD * NW * K)) * (SD * NW * K)
    n_chunks = e_pad // (NW * K)

    row = edge_index[0]
    col = edge_index[1]
    if e_pad != E:
        spread = N + jnp.arange(e_pad - E, dtype=jnp.int32) % (n_pad - N)
        row = jnp.concatenate([row, spread])
        col = jnp.concatenate([col, spread])
    row3 = row.reshape(NW, n_chunks, K)
    col3 = col.reshape(NW, n_chunks, K)
    idx4 = jnp.stack([row3, col3], axis=2)

    D2 = D_h
    W2p = jnp.zeros((D_h, D2), F32).at[:, :D_out].set(W2)

    x_pad = jnp.zeros((n_pad, D_in), F32).at[:N].set(x)
    zeros1 = jnp.zeros((ZR,), F32)
    zeros2h = jnp.zeros((ZR, D_h), F32)

    degp = _make_deg_kernel(n_pad, NC, NS, n_chunks, K)(col3, zeros1)
    degp3 = degp.reshape(NC, n_pad, 1)

    grid = n_pad // BR
    full = lambda shape: pl.BlockSpec(shape, lambda i: (0,) * len(shape))
    rowblk = lambda d: pl.BlockSpec((BR, d), lambda i: (i, 0))
    degspec = pl.BlockSpec((NC, BR, 1), lambda i: (0, i, 0))
    aggspec = lambda d: pl.BlockSpec((NC, BR, d), lambda i: (0, i, 0))

    h1p = pl.pallas_call(
        _tc1_body,
        grid=(grid,),
        in_specs=[rowblk(D_in), full((D_in, D_h)), degspec],
        out_specs=rowblk(D_h),
        out_shape=jax.ShapeDtypeStruct((n_pad, D_h), F32),
    )(x_pad, W1, degp3)

    agg1 = _make_agg_kernel(n_pad, D_h, NC, NS, n_chunks, K)(
        h1p, idx4, zeros2h)

    h2p = pl.pallas_call(
        _tc2_body,
        grid=(grid,),
        in_specs=[aggspec(D_h), rowblk(D_h), degspec, full((1, D_h)),
                  full((D_h, D2))],
        out_specs=rowblk(D2),
        out_shape=jax.ShapeDtypeStruct((n_pad, D2), F32),
    )(agg1, h1p, degp3, b1.reshape(1, D_h), W2p)

    agg2 = _make_agg_kernel(n_pad, D2, NC, NS, n_chunks, K)(
        h2p, idx4, zeros2h)

    out = pl.pallas_call(
        functools.partial(_tc3_body, D_out),
        grid=(grid,),
        in_specs=[aggspec(D2), rowblk(D2), degspec, full((1, D_out))],
        out_specs=rowblk(D_out),
        out_shape=jax.ShapeDtypeStruct((n_pad, D_out), F32),
    )(agg2, h2p, degp3, b2.reshape(1, D_out))

    return out[:N]


# speedup vs baseline: 1.0324x; 1.0324x over previous
import functools

import jax
import jax.numpy as jnp
from jax import lax
from jax.experimental import pallas as pl
from jax.experimental.pallas import tpu as pltpu
from jax.experimental.pallas import tpu_sc as plsc

F32 = jnp.float32


def _sc_info():
    try:
        info = plsc.get_sparse_core_info()
        return info.num_cores, info.num_subcores, info.num_lanes
    except Exception:
        return 2, 16, 16


def _make_deg_kernel(n_pad, nc, ns, n_chunks, K):
    NW = nc * ns
    ZR = n_pad // ns
    mesh = plsc.VectorSubcoreMesh(
        core_axis_name="c", subcore_axis_name="s", num_cores=nc,
        num_subcores=ns)

    @functools.partial(
        pl.kernel,
        out_type=jax.ShapeDtypeStruct((nc, n_pad), F32),
        mesh=mesh,
        scratch_types=[
            pltpu.VMEM((n_chunks, K), jnp.int32),
            pltpu.VMEM((K,), F32),
            pltpu.VMEM_SHARED((n_pad,), F32),
        ],
    )
    def deg_kernel(col3_hbm, zeros1_hbm, out_hbm, col2d, ones_v, deg_sh):
        cid = lax.axis_index("c")
        sid = lax.axis_index("s")
        wid = sid * nc + cid
        pltpu.sync_copy(zeros1_hbm, deg_sh.at[pl.ds(sid * ZR, ZR)])
        pltpu.sync_copy(col3_hbm.at[wid], col2d)
        for j in range(K // 16):
            ones_v[pl.ds(j * 16, 16)] = jnp.full((16,), 1.0, dtype=F32)
        plsc.subcore_barrier()

        def chunk(i, carry):
            pltpu.sync_copy(ones_v, deg_sh.at[col2d.at[i]], add=True)
            return carry

        lax.fori_loop(0, n_chunks, chunk, 0)
        plsc.subcore_barrier()
        pltpu.sync_copy(deg_sh.at[pl.ds(sid * ZR, ZR)],
                        out_hbm.at[cid, pl.ds(sid * ZR, ZR)])

    return deg_kernel


def _make_agg_kernel(n_pad, d, nc, ns, n_chunks, K):
    ZR = n_pad // ns
    mesh = plsc.VectorSubcoreMesh(
        core_axis_name="c", subcore_axis_name="s", num_cores=nc,
        num_subcores=ns)

    assert n_chunks % 2 == 0
    n2 = n_chunks // 2

    @functools.partial(
        pl.kernel,
        out_type=jax.ShapeDtypeStruct((nc, n_pad, d), F32),
        mesh=mesh,
        scratch_types=[
            pltpu.VMEM((2, K), jnp.int32),
            pltpu.VMEM((2, K), jnp.int32),
            pltpu.VMEM((K, d), F32),
            pltpu.VMEM((K, d), F32),
            pltpu.SemaphoreType.DMA,
            pltpu.SemaphoreType.DMA,
            pltpu.SemaphoreType.DMA,
            pltpu.SemaphoreType.DMA,
            pltpu.VMEM_SHARED((n_pad, d), F32),
        ],
    )
    def agg_kernel(h_hbm, idx4_hbm, zeros2_hbm, out_hbm,
                   idx_a, idx_b, buf_a, buf_b, isa, isb, gsa, gsb, agg_sh):
        cid = lax.axis_index("c")
        sid = lax.axis_index("s")
        wid = sid * nc + cid
        pltpu.sync_copy(zeros2_hbm, agg_sh.at[pl.ds(sid * ZR, ZR)])
        plsc.subcore_barrier()

        pltpu.async_copy(idx4_hbm.at[wid, 0], idx_a, isa)
        pltpu.async_copy(idx4_hbm.at[wid, 1], idx_b, isb)
        pltpu.make_async_copy(idx4_hbm.at[wid, 0], idx_a, isa).wait()
        pltpu.async_copy(h_hbm.at[idx_a.at[0]], buf_a, gsa)

        def chunk2(j, carry):
            i = 2 * j
            pltpu.make_async_copy(idx4_hbm.at[wid, i], idx_b, isb).wait()
            pltpu.make_async_copy(h_hbm.at[idx_a.at[0]], buf_a, gsa).wait()
            pltpu.async_copy(h_hbm.at[idx_b.at[0]], buf_b, gsb)
            pltpu.sync_copy(buf_a, agg_sh.at[idx_a.at[1]], add=True)

            @pl.when(j + 1 < n2)
            def _():
                pltpu.async_copy(idx4_hbm.at[wid, i + 2], idx_a, isa)

            pltpu.make_async_copy(h_hbm.at[idx_b.at[0]], buf_b, gsb).wait()

            @pl.when(j + 1 < n2)
            def _():
                pltpu.make_async_copy(
                    idx4_hbm.at[wid, i + 2], idx_a, isa).wait()
                pltpu.async_copy(h_hbm.at[idx_a.at[0]], buf_a, gsa)

            pltpu.sync_copy(buf_b, agg_sh.at[idx_b.at[1]], add=True)

            @pl.when(j + 1 < n2)
            def _():
                pltpu.async_copy(idx4_hbm.at[wid, i + 3], idx_b, isb)

            return carry

        lax.fori_loop(0, n2, chunk2, 0)
        plsc.subcore_barrier()
        pltpu.sync_copy(agg_sh.at[pl.ds(sid * ZR, ZR)],
                        out_hbm.at[cid, pl.ds(sid * ZR, ZR)])

    return agg_kernel


def _tc1_body(x_ref, w_ref, degp_ref, o_ref):
    deg = degp_ref[0] + degp_ref[1] + 1.0
    dinv = lax.rsqrt(deg)
    h = jnp.dot(x_ref[...], w_ref[...], preferred_element_type=F32)
    o_ref[...] = h * dinv


def _tc2_body(aggp_ref, hp_ref, degp_ref, b_ref, w_ref, o_ref):
    deg = degp_ref[0] + degp_ref[1] + 1.0
    dinv = lax.rsqrt(deg)
    a = aggp_ref[0] + aggp_ref[1] + hp_ref[...]
    z = a * dinv + b_ref[...]
    x1 = jnp.maximum(z, 0.0)
    o_ref[...] = jnp.dot(x1, w_ref[...], preferred_element_type=F32) * dinv


def _tc3_body(d_out, aggp_ref, hp_ref, degp_ref, b_ref, o_ref):
    deg = degp_ref[0] + degp_ref[1] + 1.0
    dinv = lax.rsqrt(deg)
    zf = (aggp_ref[0] + aggp_ref[1] + hp_ref[...]) * dinv
    z = zf[:, :d_out] + b_ref[...]
    m = jnp.max(z, axis=-1, keepdims=True)
    lse = m + jnp.log(jnp.sum(jnp.exp(z - m), axis=-1, keepdims=True))
    o_ref[...] = z - lse


def kernel(x, edge_index, W1, b1, W2, b2):
    N, D_in = x.shape
    D_h = W1.shape[1]
    D_out = W2.shape[1]
    E = edge_index.shape[1]

    NC, NS, _L = _sc_info()
    NW = NC * NS
    K = 128
    BR = 1024
    SD = 2

    n_pad = ((N + NS * 8 - 1) // (NS * 8)) * (NS * 8)
    if n_pad % BR:
        n_pad = ((n_pad + BR - 1) // BR) * BR
    ZR = n_pad // NS
    e_pad = ((E + SD * NW * K - 1) // (SD * NW * K)) * (SD * NW * K)
    n_chunks = e_pad // (NW * K)

    row = edge_index[0]
    col = edge_index[1]
    if e_pad != E:
        spread = N + jnp.arange(e_pad - E, dtype=jnp.int32) % (n_pad - N)
        row = jnp.concatenate([row, spread])
        col = jnp.concatenate([col, spread])
    row3 = row.reshape(NW, n_chunks, K)
    col3 = col.reshape(NW, n_chunks, K)
    idx4 = jnp.stack([row3, col3], axis=2)

    D2 = D_h
    W2p = jnp.zeros((D_h, D2), F32).at[:, :D_out].set(W2)

    x_pad = jnp.zeros((n_pad, D_in), F32).at[:N].set(x)
    zeros1 = jnp.zeros((ZR,), F32)
    zeros2h = jnp.zeros((ZR, D_h), F32)

    degp = _make_deg_kernel(n_pad, NC, NS, n_chunks, K)(col3, zeros1)
    degp3 = degp.reshape(NC, n_pad, 1)

    grid = n_pad // BR
    full = lambda shape: pl.BlockSpec(shape, lambda i: (0,) * len(shape))
    rowblk = lambda d: pl.BlockSpec((BR, d), lambda i: (i, 0))
    degspec = pl.BlockSpec((NC, BR, 1), lambda i: (0, i, 0))
    aggspec = lambda d: pl.BlockSpec((NC, BR, d), lambda i: (0, i, 0))

    h1p = pl.pallas_call(
        _tc1_body,
        grid=(grid,),
        in_specs=[rowblk(D_in), full((D_in, D_h)), degspec],
        out_specs=rowblk(D_h),
        out_shape=jax.ShapeDtypeStruct((n_pad, D_h), F32),
    )(x_pad, W1, degp3)

    agg1 = _make_agg_kernel(n_pad, D_h, NC, NS, n_chunks, K)(
        h1p, idx4, zeros2h)

    h2p = pl.pallas_call(
        _tc2_body,
        grid=(grid,),
        in_specs=[aggspec(D_h), rowblk(D_h), degspec, full((1, D_h)),
                  full((D_h, D2))],
        out_specs=rowblk(D2),
        out_shape=jax.ShapeDtypeStruct((n_pad, D2), F32),
    )(agg1, h1p, degp3, b1.reshape(1, D_h), W2p)

    agg2 = _make_agg_kernel(n_pad, D2, NC, NS, n_chunks, K)(
        h2p, idx4, zeros2h)

    out = pl.pallas_call(
        functools.partial(_tc3_body, D_out),
        grid=(grid,),
        in_specs=[aggspec(D2), rowblk(D2), degspec, full((1, D_out))],
        out_specs=rowblk(D_out),
        out_shape=jax.ShapeDtypeStruct((n_pad, D_out), F32),
    )(agg2, h2p, degp3, b2.reshape(1, D_out))

    return out[:N]
